# baseline (speedup 1.0000x reference)
import functools

import jax
import jax.numpy as jnp
from jax import lax
from jax.experimental import pallas as pl
from jax.experimental.pallas import tpu as pltpu
from jax.experimental.pallas import tpu_sc as plsc

SIZE = 1000000
D = 32
B = 16384
K = 5
NC = 2
NS = 16
NW = NC * NS

TCOLS = (SIZE + 127) // 128
CW = 8
NCHK = 31
RANGE = NCHK * CW
PIECE = 4096

NCAP = 1024
CCAP = 5120
WCAP = 256

SROWS = B * (K + 2) + 256
TRASH = B * (K + 2)
OFF_VI = 0
OFF_VJ = B
OFF_NG = 2 * B

_MESH = plsc.VectorSubcoreMesh(core_axis_name="c", subcore_axis_name="s")
_PARAMS = pltpu.CompilerParams(needs_layout_passes=False)


def _a_body(nodes_h, ctx_h, vi_h, vj_h, ng_h, stg_h,
            ibuf0, ibuf1, nh, nd, ch, cd, buf, whit, wdest, sidxm, rows,
            sem_idx, sem_chunk, sem_sc):
    wid = lax.axis_index("s") * NC + lax.axis_index("c")
    lane = lax.iota(jnp.int32, 16)
    start = jnp.minimum(wid * TCOLS // NW, TCOLS - RANGE)

    def mk_scan(buf_ref, hits_ref, dests_ref, cap):
        def step(k, carry):
            cur, destv = carry
            v = buf_ref[pl.ds(k * 16, 16)]
            t = v >> 7
            m = (t >= start) & (t < start + RANGE)
            cc = jnp.minimum(cur, cap)
            plsc.store_compressed(hits_ref.at[pl.ds(cc, 16)], v, mask=m)
            plsc.store_compressed(dests_ref.at[pl.ds(cc, 16)], destv, mask=m)
            n = plsc.all_reduce_population_count(m)
            return cur + n[0], destv + 16
        return step

    def scan_input(src_h, n_words, dest_off, hits_ref, dests_ref, cap, cur):
        npieces = n_words // PIECE
        destv = dest_off + lane
        cp = pltpu.async_copy(src_h.at[pl.ds(0, PIECE)], ibuf0, sem_idx)
        for p in range(npieces):
            bufp = ibuf0 if (p % 2 == 0) else ibuf1
            nxt = ibuf1 if (p % 2 == 0) else ibuf0
            cp.wait()
            if p + 1 < npieces:
                cp = pltpu.async_copy(
                    src_h.at[pl.ds((p + 1) * PIECE, PIECE)], nxt, sem_idx)
            cur, destv = lax.fori_loop(
                0, PIECE // 16, mk_scan(bufp, hits_ref, dests_ref, cap),
                (cur, destv), unroll=4)
        return cur

    with jax.named_scope("scan_phase"):
        ncnt = scan_input(vi_h, B, OFF_VI, nh, nd, NCAP, jnp.int32(0))
        ccnt = scan_input(vj_h, B, OFF_VJ, ch, cd, CCAP, jnp.int32(0))
        ccnt = scan_input(ng_h, B * K, OFF_NG, ch, cd, CCAP, ccnt)
        ncnt = jnp.minimum(ncnt, NCAP)
        ccnt = jnp.minimum(ccnt, CCAP)

    def fire_chunk(tbl_h, ci, slot):
        cb = (start + ci * CW) * 128
        for tr in range(4):
            pltpu.async_copy(
                tbl_h.at[pl.ds(tr * 8, 8), pl.ds(cb, CW * 128)],
                buf.at[pl.ds(slot * 32 + tr * 8, 8)], sem_chunk)

    def process_table(tbl_h, hits_ref, dests_ref, count):
        nvregs = (count + 15) >> 4
        fire_chunk(tbl_h, 0, 0)

        def chunk_body(ci, prev_ng):
            slot = ci & 1
            @pl.when(ci + 1 < NCHK)
            def _():
                fire_chunk(tbl_h, ci + 1, 1 - slot)
            for tr in range(4):
                pltpu.make_async_copy(
                    tbl_h.at[pl.ds(0, 8), pl.ds(0, CW * 128)],
                    buf.at[pl.ds(slot * 32 + tr * 8, 8)],
                    sem_chunk).wait()
            def dr(_, x):
                pltpu.make_async_copy(
                    stg_h.at[pl.ds(0, 128)], rows.at[pl.ds(0, 128)],
                    sem_sc).wait()
                return x
            lax.fori_loop(0, prev_ng, dr, 0)

            ct_lo = start + ci * CW

            def wl_step(k, wc):
                v = hits_ref[pl.ds(k * 16, 16)]
                dvv = dests_ref[pl.ds(k * 16, 16)]
                t = v >> 7
                m = ((k * 16 + lane) < count) & (t >= ct_lo) & (t < ct_lo + CW)
                wcc = jnp.minimum(wc, WCAP)
                plsc.store_compressed(whit.at[pl.ds(wcc, 16)], v, mask=m)
                plsc.store_compressed(wdest.at[pl.ds(wcc, 16)], dvv, mask=m)
                return wc + plsc.all_reduce_population_count(m)[0]

            wc = lax.fori_loop(0, nvregs, wl_step, jnp.int32(0))
            wc = jnp.minimum(wc, WCAP)

            for g2 in range(2):
                for vv in range(8):
                    i0 = g2 * 128 + vv * 16
                    dvv = wdest[pl.ds(i0, 16)]
                    mval = (i0 + lane) < wc
                    sidxm[g2, pl.ds(vv * 16, 16)] = jnp.where(
                        mval, dvv, TRASH + i0 + lane)

            cbase = ct_lo * 128
            rowlo = slot * 32 + lane
            rowhi = rowlo + 16

            def ex_step(g, x):
                hv = whit[pl.ds(g * 16, 16)]
                colv = jnp.clip(hv - cbase, 0, CW * 128 - 1)
                for j in range(16):
                    cv = jnp.full((16,), colv[j], jnp.int32)
                    lo = plsc.load_gather(buf, [rowlo, cv])
                    hi = plsc.load_gather(buf, [rowhi, cv])
                    rows[g * 16 + j, pl.ds(0, 16)] = lo
                    rows[g * 16 + j, pl.ds(16, 16)] = hi
                return x

            lax.fori_loop(0, (wc + 15) >> 4, ex_step, 0)

            ng2 = (wc + 127) >> 7

            def sc_step(G, x):
                pltpu.async_copy(rows.at[pl.ds(G * 128, 128)],
                                 stg_h.at[sidxm.at[G]], sem_sc)
                return x

            lax.fori_loop(0, ng2, sc_step, 0)
            return ng2

        last_ng = lax.fori_loop(0, NCHK, chunk_body, jnp.int32(0))

        def drf(_, x):
            pltpu.make_async_copy(
                stg_h.at[pl.ds(0, 128)], rows.at[pl.ds(0, 128)],
                sem_sc).wait()
            return x
        lax.fori_loop(0, last_ng, drf, 0)

    with jax.named_scope("nodes_stream"):
        process_table(nodes_h, nh, nd, ncnt)
    with jax.named_scope("ctx_stream"):
        process_table(ctx_h, ch, cd, ccnt)


_kernel_a = functools.partial(
    pl.kernel,
    mesh=_MESH,
    out_type=jax.ShapeDtypeStruct((SROWS, 128), jnp.float32),
    scratch_types=[
        pltpu.VMEM((PIECE,), jnp.int32),
        pltpu.VMEM((PIECE,), jnp.int32),
        pltpu.VMEM((NCAP + 16,), jnp.int32),
        pltpu.VMEM((NCAP + 16,), jnp.int32),
        pltpu.VMEM((CCAP + 16,), jnp.int32),
        pltpu.VMEM((CCAP + 16,), jnp.int32),
        pltpu.VMEM((64, CW * 128), jnp.float32),
        pltpu.VMEM((WCAP + 16,), jnp.int32),
        pltpu.VMEM((WCAP + 16,), jnp.int32),
        pltpu.VMEM((2, 128), jnp.int32),
        pltpu.VMEM((WCAP, 128), jnp.float32),
        pltpu.SemaphoreType.DMA,
        pltpu.SemaphoreType.DMA,
        pltpu.SemaphoreType.DMA,
    ],
    compiler_params=_PARAMS,
)(_a_body)


def _b_body(stg_h, out_h, vib, vjb, ngb, dots, accv, sem):
    wid = lax.axis_index("s") * NC + lax.axis_index("c")
    lane = lax.iota(jnp.int32, 16)
    last = lane == 15
    SB = 64
    acc = jnp.zeros((16,), jnp.float32)

    for s in range(512 // SB):
        base = wid * 512 + s * SB
        c1 = pltpu.async_copy(
            stg_h.at[pl.ds(OFF_VI + base, SB), pl.ds(0, 32)], vib, sem)
        c2 = pltpu.async_copy(
            stg_h.at[pl.ds(OFF_VJ + base, SB), pl.ds(0, 32)], vjb, sem)
        c3 = pltpu.async_copy(
            stg_h.at[pl.ds(OFF_NG + base * K, SB * K), pl.ds(0, 32)], ngb, sem)
        c1.wait()
        c2.wait()
        c3.wait()

        def bstep(b, ivec):
            vi0 = vib[b, pl.ds(0, 16)]
            vi1 = vib[b, pl.ds(16, 16)]
            vj0 = vjb[b, pl.ds(0, 16)]
            vj1 = vjb[b, pl.ds(16, 16)]
            cpos = plsc.cumsum(vi0 * vj0 + vi1 * vj1)
            plsc.store_scatter(dots, [ivec], cpos, mask=last)
            nvi0 = -vi0
            nvi1 = -vi1
            for k in range(K):
                n0 = ngb[b * K + k, pl.ds(0, 16)]
                n1 = ngb[b * K + k, pl.ds(16, 16)]
                cneg = plsc.cumsum(nvi0 * n0 + nvi1 * n1)
                plsc.store_scatter(dots, [ivec + (k + 1)], cneg, mask=last)
            return ivec + (K + 1)

        lax.fori_loop(0, SB, bstep, jnp.zeros((16,), jnp.int32))

        def sstep(j, a):
            dv = dots[pl.ds(j * 16, 16)]
            return a + 1.0 / (1.0 + jnp.exp(-dv))

        acc = lax.fori_loop(0, SB * (K + 1) // 16, sstep, acc)

    accv[pl.ds(0, 16)] = acc
    for j in range(1, 8):
        accv[pl.ds(j * 16, 16)] = jnp.zeros((16,), jnp.float32)
    pltpu.sync_copy(accv, out_h.at[wid])


_kernel_b = functools.partial(
    pl.kernel,
    mesh=_MESH,
    out_type=jax.ShapeDtypeStruct((NW, 128), jnp.float32),
    scratch_types=[
        pltpu.VMEM((64, 32), jnp.float32),
        pltpu.VMEM((64, 32), jnp.float32),
        pltpu.VMEM((64 * K, 32), jnp.float32),
        pltpu.VMEM((64 * (K + 1),), jnp.float32),
        pltpu.VMEM((128,), jnp.float32),
        pltpu.SemaphoreType.DMA,
    ],
    compiler_params=pltpu.CompilerParams(
        needs_layout_passes=False, use_tc_tiling_on_sc=False),
)(_b_body)


def kernel(v_i, v_j, negsamples, nodes_embeddings, contextnodes_embeddings):
    nodes_t = nodes_embeddings.T
    ctx_t = contextnodes_embeddings.T
    vi = v_i.astype(jnp.int32)
    vj = v_j.astype(jnp.int32)
    ng = negsamples.astype(jnp.int32).reshape(-1)
    staging = _kernel_a(nodes_t, ctx_t, vi, vj, ng)
    partials = _kernel_b(staging)
    return -(jnp.sum(partials) / B)


# speedup vs baseline: 1.3819x; 1.3819x over previous
import functools

import jax
import jax.numpy as jnp
from jax import lax
from jax.experimental import pallas as pl
from jax.experimental.pallas import tpu as pltpu
from jax.experimental.pallas import tpu_sc as plsc

SIZE = 1000000
D = 32
B = 16384
K = 5
NC = 2
NS = 16
NW = NC * NS

TCOLS = (SIZE + 127) // 128
CW = 8
NCHK = 31
RANGE = NCHK * CW
PIECE = 4096

NCAP = 1024
CCAP = 5120
WCAP = 256

SROWS = B * (K + 2) + NW * 16
TRASH = B * (K + 2)
OFF_VI = 0
OFF_VJ = B
OFF_NG = 2 * B

_MESH = plsc.VectorSubcoreMesh(core_axis_name="c", subcore_axis_name="s")
_PARAMS = pltpu.CompilerParams(needs_layout_passes=False)


def _a_body(nodes_h, ctx_h, vi_h, vj_h, ng_h, stg_h,
            ibuf0, ibuf1, nh, nd, ch, cd, buf, whit, wdest, sidxm, rows,
            sem_idx, sem_chunk, sem_sc):
    wid = lax.axis_index("s") * NC + lax.axis_index("c")
    lane = lax.iota(jnp.int32, 16)
    start = jnp.minimum(wid * TCOLS // NW, TCOLS - RANGE)

    def mk_scan(buf_ref, hits_ref, dests_ref, cap):
        def step(k, carry):
            cur, destv = carry
            v = buf_ref[pl.ds(k * 16, 16)]
            t = v >> 7
            m = (t >= start) & (t < start + RANGE)
            cc = jnp.minimum(cur, cap)
            plsc.store_compressed(hits_ref.at[pl.ds(cc, 16)], v, mask=m)
            plsc.store_compressed(dests_ref.at[pl.ds(cc, 16)], destv, mask=m)
            n = plsc.all_reduce_population_count(m)
            return cur + n[0], destv + 16
        return step

    def scan_input(src_h, n_words, dest_off, hits_ref, dests_ref, cap, cur):
        npieces = n_words // PIECE
        destv = dest_off + lane
        cp = pltpu.async_copy(src_h.at[pl.ds(0, PIECE)], ibuf0, sem_idx)
        for p in range(npieces):
            bufp = ibuf0 if (p % 2 == 0) else ibuf1
            nxt = ibuf1 if (p % 2 == 0) else ibuf0
            cp.wait()
            if p + 1 < npieces:
                cp = pltpu.async_copy(
                    src_h.at[pl.ds((p + 1) * PIECE, PIECE)], nxt, sem_idx)
            cur, destv = lax.fori_loop(
                0, PIECE // 16, mk_scan(bufp, hits_ref, dests_ref, cap),
                (cur, destv), unroll=4)
        return cur

    with jax.named_scope("scan_phase"):
        ncnt = scan_input(vi_h, B, OFF_VI, nh, nd, NCAP, jnp.int32(0))
        ccnt = scan_input(vj_h, B, OFF_VJ, ch, cd, CCAP, jnp.int32(0))
        ccnt = scan_input(ng_h, B * K, OFF_NG, ch, cd, CCAP, ccnt)
        ncnt = jnp.minimum(ncnt, NCAP)
        ccnt = jnp.minimum(ccnt, CCAP)

    def fire_chunk(tbl_h, ci, slot):
        cb = (start + ci * CW) * 128
        for tr in range(4):
            pltpu.async_copy(
                tbl_h.at[pl.ds(tr * 8, 8), pl.ds(cb, CW * 128)],
                buf.at[pl.ds(slot * 32 + tr * 8, 8)], sem_chunk)

    def process_table(tbl_h, hits_ref, dests_ref, count):
        nvregs = (count + 15) >> 4
        fire_chunk(tbl_h, 0, 0)

        def chunk_body(ci, prev_ng):
            slot = ci & 1
            @pl.when(ci + 1 < NCHK)
            def _():
                fire_chunk(tbl_h, ci + 1, 1 - slot)
            for tr in range(4):
                pltpu.make_async_copy(
                    tbl_h.at[pl.ds(0, 8), pl.ds(0, CW * 128)],
                    buf.at[pl.ds(slot * 32 + tr * 8, 8)],
                    sem_chunk).wait()
            def dr(_, x):
                pltpu.make_async_copy(
                    stg_h.at[pl.ds(0, 16)], rows.at[pl.ds(0, 16)],
                    sem_sc).wait()
                return x
            lax.fori_loop(0, prev_ng, dr, 0)

            ct_lo = start + ci * CW

            def wl_step(k, wc):
                v = hits_ref[pl.ds(k * 16, 16)]
                dvv = dests_ref[pl.ds(k * 16, 16)]
                t = v >> 7
                m = ((k * 16 + lane) < count) & (t >= ct_lo) & (t < ct_lo + CW)
                wcc = jnp.minimum(wc, WCAP)
                plsc.store_compressed(whit.at[pl.ds(wcc, 16)], v, mask=m)
                plsc.store_compressed(wdest.at[pl.ds(wcc, 16)], dvv, mask=m)
                return wc + plsc.all_reduce_population_count(m)[0]

            wc = lax.fori_loop(0, nvregs, wl_step, jnp.int32(0))
            wc = jnp.minimum(wc, WCAP)

            for g2 in range(16):
                i0 = g2 * 16
                dvv = wdest[pl.ds(i0, 16)]
                mval = (i0 + lane) < wc
                sidxm[g2, pl.ds(0, 16)] = jnp.where(
                    mval, dvv, TRASH + wid * 16 + lane)

            cbase = ct_lo * 128
            rowlo = slot * 32 + lane
            rowhi = rowlo + 16

            def ex_step(g, x):
                hv = whit[pl.ds(g * 16, 16)]
                colv = jnp.clip(hv - cbase, 0, CW * 128 - 1)
                for j in range(16):
                    cv = jnp.full((16,), colv[j], jnp.int32)
                    lo = plsc.load_gather(buf, [rowlo, cv])
                    hi = plsc.load_gather(buf, [rowhi, cv])
                    rows[g * 16 + j, pl.ds(0, 16)] = lo
                    rows[g * 16 + j, pl.ds(16, 16)] = hi
                return x

            lax.fori_loop(0, (wc + 15) >> 4, ex_step, 0)

            ng2 = (wc + 15) >> 4

            def sc_step(G, x):
                pltpu.async_copy(rows.at[pl.ds(G * 16, 16)],
                                 stg_h.at[sidxm.at[G]], sem_sc)
                return x

            lax.fori_loop(0, ng2, sc_step, 0)
            return ng2

        last_ng = lax.fori_loop(0, NCHK, chunk_body, jnp.int32(0))

        def drf(_, x):
            pltpu.make_async_copy(
                stg_h.at[pl.ds(0, 16)], rows.at[pl.ds(0, 16)],
                sem_sc).wait()
            return x
        lax.fori_loop(0, last_ng, drf, 0)

    with jax.named_scope("nodes_stream"):
        process_table(nodes_h, nh, nd, ncnt)
    with jax.named_scope("ctx_stream"):
        process_table(ctx_h, ch, cd, ccnt)


_kernel_a = functools.partial(
    pl.kernel,
    mesh=_MESH,
    out_type=jax.ShapeDtypeStruct((SROWS, 128), jnp.float32),
    scratch_types=[
        pltpu.VMEM((PIECE,), jnp.int32),
        pltpu.VMEM((PIECE,), jnp.int32),
        pltpu.VMEM((NCAP + 16,), jnp.int32),
        pltpu.VMEM((NCAP + 16,), jnp.int32),
        pltpu.VMEM((CCAP + 16,), jnp.int32),
        pltpu.VMEM((CCAP + 16,), jnp.int32),
        pltpu.VMEM((64, CW * 128), jnp.float32),
        pltpu.VMEM((WCAP + 16,), jnp.int32),
        pltpu.VMEM((WCAP + 16,), jnp.int32),
        pltpu.VMEM((16, 16), jnp.int32),
        pltpu.VMEM((WCAP, 128), jnp.float32),
        pltpu.SemaphoreType.DMA,
        pltpu.SemaphoreType.DMA,
        pltpu.SemaphoreType.DMA,
    ],
    compiler_params=_PARAMS,
)(_a_body)


def _b_body(stg_h, out_h, vib, vjb, ngb, dots, accv, sem):
    wid = lax.axis_index("s") * NC + lax.axis_index("c")
    lane = lax.iota(jnp.int32, 16)
    last = lane == 15
    SB = 64
    acc = jnp.zeros((16,), jnp.float32)

    for s in range(512 // SB):
        base = wid * 512 + s * SB
        c1 = pltpu.async_copy(
            stg_h.at[pl.ds(OFF_VI + base, SB), pl.ds(0, 32)], vib, sem)
        c2 = pltpu.async_copy(
            stg_h.at[pl.ds(OFF_VJ + base, SB), pl.ds(0, 32)], vjb, sem)
        c3 = pltpu.async_copy(
            stg_h.at[pl.ds(OFF_NG + base * K, SB * K), pl.ds(0, 32)], ngb, sem)
        c1.wait()
        c2.wait()
        c3.wait()

        def bstep(b, ivec):
            vi0 = vib[b, pl.ds(0, 16)]
            vi1 = vib[b, pl.ds(16, 16)]
            vj0 = vjb[b, pl.ds(0, 16)]
            vj1 = vjb[b, pl.ds(16, 16)]
            cpos = plsc.cumsum(vi0 * vj0 + vi1 * vj1)
            plsc.store_scatter(dots, [ivec], cpos, mask=last)
            nvi0 = -vi0
            nvi1 = -vi1
            for k in range(K):
                n0 = ngb[b * K + k, pl.ds(0, 16)]
                n1 = ngb[b * K + k, pl.ds(16, 16)]
                cneg = plsc.cumsum(nvi0 * n0 + nvi1 * n1)
                plsc.store_scatter(dots, [ivec + (k + 1)], cneg, mask=last)
            return ivec + (K + 1)

        lax.fori_loop(0, SB, bstep, jnp.zeros((16,), jnp.int32))

        def sstep(j, a):
            dv = dots[pl.ds(j * 16, 16)]
            return a + 1.0 / (1.0 + jnp.exp(-dv))

        acc = lax.fori_loop(0, SB * (K + 1) // 16, sstep, acc)

    accv[pl.ds(0, 16)] = acc
    for j in range(1, 8):
        accv[pl.ds(j * 16, 16)] = jnp.zeros((16,), jnp.float32)
    pltpu.sync_copy(accv, out_h.at[wid])


_kernel_b = functools.partial(
    pl.kernel,
    mesh=_MESH,
    out_type=jax.ShapeDtypeStruct((NW, 128), jnp.float32),
    scratch_types=[
        pltpu.VMEM((64, 32), jnp.float32),
        pltpu.VMEM((64, 32), jnp.float32),
        pltpu.VMEM((64 * K, 32), jnp.float32),
        pltpu.VMEM((64 * (K + 1),), jnp.float32),
        pltpu.VMEM((128,), jnp.float32),
        pltpu.SemaphoreType.DMA,
    ],
    compiler_params=pltpu.CompilerParams(
        needs_layout_passes=False, use_tc_tiling_on_sc=False),
)(_b_body)


def kernel(v_i, v_j, negsamples, nodes_embeddings, contextnodes_embeddings):
    nodes_t = nodes_embeddings.T
    ctx_t = contextnodes_embeddings.T
    vi = v_i.astype(jnp.int32)
    vj = v_j.astype(jnp.int32)
    ng = negsamples.astype(jnp.int32).reshape(-1)
    staging = _kernel_a(nodes_t, ctx_t, vi, vj, ng)
    partials = _kernel_b(staging)
    return -(jnp.sum(partials) / B)


# speedup vs baseline: 1.4867x; 1.0759x over previous
import functools

import jax
import jax.numpy as jnp
from jax import lax
from jax.experimental import pallas as pl
from jax.experimental.pallas import tpu as pltpu
from jax.experimental.pallas import tpu_sc as plsc

SIZE = 1000000
D = 32
B = 16384
K = 5
NC = 2
NS = 16
NW = NC * NS

TCOLS = (SIZE + 127) // 128
CW = 8
NCHK = 31
RANGE = NCHK * CW
PIECE = 2048

NHALF = 512
CHALF = 2560
WCAP = 256

SROWS = B * (K + 2) + NW * 16
TRASH = B * (K + 2)
OFF_VI = 0
OFF_VJ = B
OFF_NG = 2 * B

_MESH = plsc.VectorSubcoreMesh(core_axis_name="c", subcore_axis_name="s")
_PARAMS = pltpu.CompilerParams(needs_layout_passes=False)


def _a_body(nodes_h, ctx_h, vi_h, vj_h, ng_h, stg_h,
            ibuf0, ibuf1, ibuf2, ibuf3, nh, nd, ch, cd, buf, whit, wdest,
            sidxm, rows, sem_idx, sem_chunk, sem_sc):
    wid = lax.axis_index("s") * NC + lax.axis_index("c")
    lane = lax.iota(jnp.int32, 16)
    start = jnp.minimum(wid * TCOLS // NW, TCOLS - RANGE)

    def mk_scan2(bufA, bufB, hits_ref, dests_ref, half, boff):
        def step(k, carry):
            curA, curB, dvA, dvB = carry
            vA = bufA[pl.ds(k * 16, 16)]
            vB = bufB[pl.ds(k * 16, 16)]
            tA = vA >> 7
            tB = vB >> 7
            mA = (tA >= start) & (tA < start + RANGE)
            mB = (tB >= start) & (tB < start + RANGE)
            ccA = jnp.minimum(curA, half)
            ccB = boff + jnp.minimum(curB, half)
            plsc.store_compressed(hits_ref.at[pl.ds(ccA, 16)], vA, mask=mA)
            plsc.store_compressed(dests_ref.at[pl.ds(ccA, 16)], dvA, mask=mA)
            plsc.store_compressed(hits_ref.at[pl.ds(ccB, 16)], vB, mask=mB)
            plsc.store_compressed(dests_ref.at[pl.ds(ccB, 16)], dvB, mask=mB)
            nA = plsc.all_reduce_population_count(mA)
            nB = plsc.all_reduce_population_count(mB)
            return curA + nA[0], curB + nB[0], dvA + 16, dvB + 16
        return step

    def scan_input(src_h, n_words, dest_off, hits_ref, dests_ref, half,
                   curA, curB):
        npairs = n_words // (2 * PIECE)
        boff = half + 16
        cps = [pltpu.async_copy(src_h.at[pl.ds(0, PIECE)], ibuf0, sem_idx),
               pltpu.async_copy(src_h.at[pl.ds(PIECE, PIECE)], ibuf1, sem_idx)]
        for q in range(npairs):
            bA = ibuf0 if (q % 2 == 0) else ibuf2
            bB = ibuf1 if (q % 2 == 0) else ibuf3
            nA = ibuf2 if (q % 2 == 0) else ibuf0
            nB = ibuf3 if (q % 2 == 0) else ibuf1
            cps[0].wait()
            cps[1].wait()
            if q + 1 < npairs:
                w0 = (q + 1) * 2 * PIECE
                cps = [pltpu.async_copy(src_h.at[pl.ds(w0, PIECE)], nA,
                                        sem_idx),
                       pltpu.async_copy(src_h.at[pl.ds(w0 + PIECE, PIECE)],
                                        nB, sem_idx)]
            dvA = dest_off + q * 2 * PIECE + lane
            dvB = dvA + PIECE
            curA, curB, _, _ = lax.fori_loop(
                0, PIECE // 16,
                mk_scan2(bA, bB, hits_ref, dests_ref, half, boff),
                (curA, curB, dvA, dvB), unroll=2)
        return curA, curB

    with jax.named_scope("scan_phase"):
        ncA, ncB = scan_input(vi_h, B, OFF_VI, nh, nd, NHALF,
                              jnp.int32(0), jnp.int32(0))
        ccA, ccB = scan_input(vj_h, B, OFF_VJ, ch, cd, CHALF,
                              jnp.int32(0), jnp.int32(0))
        ccA, ccB = scan_input(ng_h, B * K, OFF_NG, ch, cd, CHALF, ccA, ccB)
        ncA = jnp.minimum(ncA, NHALF)
        ncB = jnp.minimum(ncB, NHALF)
        ccA = jnp.minimum(ccA, CHALF)
        ccB = jnp.minimum(ccB, CHALF)

    def fire_chunk(tbl_h, ci, slot):
        cb = (start + ci * CW) * 128
        for tr in range(4):
            pltpu.async_copy(
                tbl_h.at[pl.ds(tr * 8, 8), pl.ds(cb, CW * 128)],
                buf.at[pl.ds(slot * 32 + tr * 8, 8)], sem_chunk)

    def process_table(tbl_h, hits_ref, dests_ref, cntA, cntB, half):
        nvA = (cntA + 15) >> 4
        nvB = (cntB + 15) >> 4
        b16 = (half + 16) // 16
        fire_chunk(tbl_h, 0, 0)

        def chunk_body(ci, prev_ng):
            slot = ci & 1
            @pl.when(ci + 1 < NCHK)
            def _():
                fire_chunk(tbl_h, ci + 1, 1 - slot)
            for tr in range(4):
                pltpu.make_async_copy(
                    tbl_h.at[pl.ds(0, 8), pl.ds(0, CW * 128)],
                    buf.at[pl.ds(slot * 32 + tr * 8, 8)],
                    sem_chunk).wait()
            def dr(_, x):
                pltpu.make_async_copy(
                    stg_h.at[pl.ds(0, 16)], rows.at[pl.ds(0, 16)],
                    sem_sc).wait()
                return x
            lax.fori_loop(0, prev_ng, dr, 0)

            ct_lo = start + ci * CW

            def mk_wl(base16, cnt):
                def wl_step(k, wc):
                    v = hits_ref[pl.ds((base16 + k) * 16, 16)]
                    dvv = dests_ref[pl.ds((base16 + k) * 16, 16)]
                    t = v >> 7
                    m = (((k * 16 + lane) < cnt)
                         & (t >= ct_lo) & (t < ct_lo + CW))
                    wcc = jnp.minimum(wc, WCAP)
                    plsc.store_compressed(whit.at[pl.ds(wcc, 16)], v, mask=m)
                    plsc.store_compressed(wdest.at[pl.ds(wcc, 16)], dvv,
                                          mask=m)
                    return wc + plsc.all_reduce_population_count(m)[0]
                return wl_step

            wc = lax.fori_loop(0, nvA, mk_wl(0, cntA), jnp.int32(0))
            wc = lax.fori_loop(0, nvB, mk_wl(b16, cntB), wc)
            wc = jnp.minimum(wc, WCAP)

            for g2 in range(16):
                i0 = g2 * 16
                dvv = wdest[pl.ds(i0, 16)]
                mval = (i0 + lane) < wc
                sidxm[g2, pl.ds(0, 16)] = jnp.where(
                    mval, dvv, TRASH + wid * 16 + lane)

            cbase = ct_lo * 128
            rowlo = slot * 32 + lane
            rowhi = rowlo + 16

            def ex_step(g, x):
                hv = whit[pl.ds(g * 16, 16)]
                colv = jnp.clip(hv - cbase, 0, CW * 128 - 1)
                for j in range(16):
                    cv = jnp.full((16,), colv[j], jnp.int32)
                    lo = plsc.load_gather(buf, [rowlo, cv])
                    hi = plsc.load_gather(buf, [rowhi, cv])
                    rows[g * 16 + j, pl.ds(0, 16)] = lo
                    rows[g * 16 + j, pl.ds(16, 16)] = hi
                return x

            lax.fori_loop(0, (wc + 15) >> 4, ex_step, 0)

            ng2 = (wc + 15) >> 4

            def sc_step(G, x):
                pltpu.async_copy(rows.at[pl.ds(G * 16, 16)],
                                 stg_h.at[sidxm.at[G]], sem_sc)
                return x

            lax.fori_loop(0, ng2, sc_step, 0)
            return ng2

        last_ng = lax.fori_loop(0, NCHK, chunk_body, jnp.int32(0))

        def drf(_, x):
            pltpu.make_async_copy(
                stg_h.at[pl.ds(0, 16)], rows.at[pl.ds(0, 16)],
                sem_sc).wait()
            return x
        lax.fori_loop(0, last_ng, drf, 0)

    with jax.named_scope("nodes_stream"):
        process_table(nodes_h, nh, nd, ncA, ncB, NHALF)
    with jax.named_scope("ctx_stream"):
        process_table(ctx_h, ch, cd, ccA, ccB, CHALF)


_kernel_a = functools.partial(
    pl.kernel,
    mesh=_MESH,
    out_type=jax.ShapeDtypeStruct((SROWS, 128), jnp.float32),
    scratch_types=[
        pltpu.VMEM((PIECE,), jnp.int32),
        pltpu.VMEM((PIECE,), jnp.int32),
        pltpu.VMEM((PIECE,), jnp.int32),
        pltpu.VMEM((PIECE,), jnp.int32),
        pltpu.VMEM((2 * NHALF + 32,), jnp.int32),
        pltpu.VMEM((2 * NHALF + 32,), jnp.int32),
        pltpu.VMEM((2 * CHALF + 32,), jnp.int32),
        pltpu.VMEM((2 * CHALF + 32,), jnp.int32),
        pltpu.VMEM((64, CW * 128), jnp.float32),
        pltpu.VMEM((WCAP + 16,), jnp.int32),
        pltpu.VMEM((WCAP + 16,), jnp.int32),
        pltpu.VMEM((16, 16), jnp.int32),
        pltpu.VMEM((WCAP, 128), jnp.float32),
        pltpu.SemaphoreType.DMA,
        pltpu.SemaphoreType.DMA,
        pltpu.SemaphoreType.DMA,
    ],
    compiler_params=_PARAMS,
)(_a_body)


def _b_body(stg_h, out_h, vib, vjb, ngb, dots, accv, sem):
    wid = lax.axis_index("s") * NC + lax.axis_index("c")
    lane = lax.iota(jnp.int32, 16)
    last = lane == 15
    SB = 64
    acc = jnp.zeros((16,), jnp.float32)

    for s in range(512 // SB):
        base = wid * 512 + s * SB
        c1 = pltpu.async_copy(
            stg_h.at[pl.ds(OFF_VI + base, SB), pl.ds(0, 32)], vib, sem)
        c2 = pltpu.async_copy(
            stg_h.at[pl.ds(OFF_VJ + base, SB), pl.ds(0, 32)], vjb, sem)
        c3 = pltpu.async_copy(
            stg_h.at[pl.ds(OFF_NG + base * K, SB * K), pl.ds(0, 32)], ngb, sem)
        c1.wait()
        c2.wait()
        c3.wait()

        def bstep(b, ivec):
            vi0 = vib[b, pl.ds(0, 16)]
            vi1 = vib[b, pl.ds(16, 16)]
            vj0 = vjb[b, pl.ds(0, 16)]
            vj1 = vjb[b, pl.ds(16, 16)]
            cpos = plsc.cumsum(vi0 * vj0 + vi1 * vj1)
            plsc.store_scatter(dots, [ivec], cpos, mask=last)
            nvi0 = -vi0
            nvi1 = -vi1
            for k in range(K):
                n0 = ngb[b * K + k, pl.ds(0, 16)]
                n1 = ngb[b * K + k, pl.ds(16, 16)]
                cneg = plsc.cumsum(nvi0 * n0 + nvi1 * n1)
                plsc.store_scatter(dots, [ivec + (k + 1)], cneg, mask=last)
            return ivec + (K + 1)

        lax.fori_loop(0, SB, bstep, jnp.zeros((16,), jnp.int32))

        def sstep(j, a):
            dv = dots[pl.ds(j * 16, 16)]
            return a + 1.0 / (1.0 + jnp.exp(-dv))

        acc = lax.fori_loop(0, SB * (K + 1) // 16, sstep, acc)

    accv[pl.ds(0, 16)] = acc
    for j in range(1, 8):
        accv[pl.ds(j * 16, 16)] = jnp.zeros((16,), jnp.float32)
    pltpu.sync_copy(accv, out_h.at[wid])


_kernel_b = functools.partial(
    pl.kernel,
    mesh=_MESH,
    out_type=jax.ShapeDtypeStruct((NW, 128), jnp.float32),
    scratch_types=[
        pltpu.VMEM((64, 32), jnp.float32),
        pltpu.VMEM((64, 32), jnp.float32),
        pltpu.VMEM((64 * K, 32), jnp.float32),
        pltpu.VMEM((64 * (K + 1),), jnp.float32),
        pltpu.VMEM((128,), jnp.float32),
        pltpu.SemaphoreType.DMA,
    ],
    compiler_params=pltpu.CompilerParams(
        needs_layout_passes=False, use_tc_tiling_on_sc=False),
)(_b_body)


def kernel(v_i, v_j, negsamples, nodes_embeddings, contextnodes_embeddings):
    nodes_t = nodes_embeddings.T
    ctx_t = contextnodes_embeddings.T
    vi = v_i.astype(jnp.int32)
    vj = v_j.astype(jnp.int32)
    ng = negsamples.astype(jnp.int32).reshape(-1)
    staging = _kernel_a(nodes_t, ctx_t, vi, vj, ng)
    partials = _kernel_b(staging)
    return -(jnp.sum(partials) / B)


# speedup vs baseline: 1.5296x; 1.0288x over previous
import functools

import jax
import jax.numpy as jnp
from jax import lax
from jax.experimental import pallas as pl
from jax.experimental.pallas import tpu as pltpu
from jax.experimental.pallas import tpu_sc as plsc

SIZE = 1000000
D = 32
B = 16384
K = 5
NC = 2
NS = 16
NW = NC * NS

TCOLS = (SIZE + 127) // 128
CW = 8
NCHK = 31
RANGE = NCHK * CW
PIECE = 2048

NHALF = 512
CHALF = 2560
WCAP = 256

SROWS = B * (K + 2) + NW * 16
TRASH = B * (K + 2)
OFF_VI = 0
OFF_VJ = B
OFF_NG = 2 * B

_MESH = plsc.VectorSubcoreMesh(core_axis_name="c", subcore_axis_name="s")
_PARAMS = pltpu.CompilerParams(needs_layout_passes=False)


def _a_body(nodes_h, ctx_h, vi_h, vj_h, ng_h, stg_h,
            ibuf0, ibuf1, ibuf2, ibuf3, nh, nd, ch, cd, buf, whit, wdest,
            sidxm, rows, sem_idx, sem_chunk, sem_sc):
    wid = lax.axis_index("s") * NC + lax.axis_index("c")
    lane = lax.iota(jnp.int32, 16)
    start = jnp.minimum(wid * TCOLS // NW, TCOLS - RANGE)

    def mk_scan2(bufA, bufB, hits_ref, dests_ref, half, boff):
        def step(k, carry):
            curA, curB, dvA, dvB = carry
            vA = bufA[pl.ds(k * 16, 16)]
            vB = bufB[pl.ds(k * 16, 16)]
            tA = vA >> 7
            tB = vB >> 7
            mA = (tA >= start) & (tA < start + RANGE)
            mB = (tB >= start) & (tB < start + RANGE)
            ccA = jnp.minimum(curA, half)
            ccB = boff + jnp.minimum(curB, half)
            plsc.store_compressed(hits_ref.at[pl.ds(ccA, 16)], vA, mask=mA)
            plsc.store_compressed(dests_ref.at[pl.ds(ccA, 16)], dvA, mask=mA)
            plsc.store_compressed(hits_ref.at[pl.ds(ccB, 16)], vB, mask=mB)
            plsc.store_compressed(dests_ref.at[pl.ds(ccB, 16)], dvB, mask=mB)
            nA = plsc.all_reduce_population_count(mA)
            nB = plsc.all_reduce_population_count(mB)
            return curA + nA[0], curB + nB[0], dvA + 16, dvB + 16
        return step

    def scan_input(src_h, n_words, dest_off, hits_ref, dests_ref, half,
                   curA, curB):
        npairs = n_words // (2 * PIECE)
        boff = half + 16
        cps = [pltpu.async_copy(src_h.at[pl.ds(0, PIECE)], ibuf0, sem_idx),
               pltpu.async_copy(src_h.at[pl.ds(PIECE, PIECE)], ibuf1, sem_idx)]
        for q in range(npairs):
            bA = ibuf0 if (q % 2 == 0) else ibuf2
            bB = ibuf1 if (q % 2 == 0) else ibuf3
            nA = ibuf2 if (q % 2 == 0) else ibuf0
            nB = ibuf3 if (q % 2 == 0) else ibuf1
            cps[0].wait()
            cps[1].wait()
            if q + 1 < npairs:
                w0 = (q + 1) * 2 * PIECE
                cps = [pltpu.async_copy(src_h.at[pl.ds(w0, PIECE)], nA,
                                        sem_idx),
                       pltpu.async_copy(src_h.at[pl.ds(w0 + PIECE, PIECE)],
                                        nB, sem_idx)]
            dvA = dest_off + q * 2 * PIECE + lane
            dvB = dvA + PIECE
            curA, curB, _, _ = lax.fori_loop(
                0, PIECE // 16,
                mk_scan2(bA, bB, hits_ref, dests_ref, half, boff),
                (curA, curB, dvA, dvB), unroll=2)
        return curA, curB

    with jax.named_scope("scan_phase"):
        ncA, ncB = scan_input(vi_h, B, OFF_VI, nh, nd, NHALF,
                              jnp.int32(0), jnp.int32(0))
        ccA, ccB = scan_input(vj_h, B, OFF_VJ, ch, cd, CHALF,
                              jnp.int32(0), jnp.int32(0))
        ccA, ccB = scan_input(ng_h, B * K, OFF_NG, ch, cd, CHALF, ccA, ccB)
        ncA = jnp.minimum(ncA, NHALF)
        ncB = jnp.minimum(ncB, NHALF)
        ccA = jnp.minimum(ccA, CHALF)
        ccB = jnp.minimum(ccB, CHALF)

    def fire_chunk(tbl_h, ci, slot):
        cb = (start + ci * CW) * 128
        for tr in range(4):
            pltpu.async_copy(
                tbl_h.at[pl.ds(tr * 8, 8), pl.ds(cb, CW * 128)],
                buf.at[pl.ds(slot * 32 + tr * 8, 8)], sem_chunk)

    def process_table(tbl_h, hits_ref, dests_ref, cntA, cntB, half):
        nvA = (cntA + 15) >> 4
        nvB = (cntB + 15) >> 4
        b16 = (half + 16) // 16
        fire_chunk(tbl_h, 0, 0)

        def chunk_body(ci, prev_ng):
            slot = ci & 1
            @pl.when(ci + 1 < NCHK)
            def _():
                fire_chunk(tbl_h, ci + 1, 1 - slot)
            for tr in range(4):
                pltpu.make_async_copy(
                    tbl_h.at[pl.ds(0, 8), pl.ds(0, CW * 128)],
                    buf.at[pl.ds(slot * 32 + tr * 8, 8)],
                    sem_chunk).wait()
            def dr(_, x):
                pltpu.make_async_copy(
                    stg_h.at[pl.ds(0, 16)], rows.at[pl.ds(0, 16)],
                    sem_sc).wait()
                return x
            lax.fori_loop(0, prev_ng, dr, 0)

            ct_lo = start + ci * CW

            def mk_wl(base16, cnt):
                def wl_step(k, wc):
                    v = hits_ref[pl.ds((base16 + k) * 16, 16)]
                    dvv = dests_ref[pl.ds((base16 + k) * 16, 16)]
                    t = v >> 7
                    m = (((k * 16 + lane) < cnt)
                         & (t >= ct_lo) & (t < ct_lo + CW))
                    wcc = jnp.minimum(wc, WCAP)
                    plsc.store_compressed(whit.at[pl.ds(wcc, 16)], v, mask=m)
                    plsc.store_compressed(wdest.at[pl.ds(wcc, 16)], dvv,
                                          mask=m)
                    return wc + plsc.all_reduce_population_count(m)[0]
                return wl_step

            wc = lax.fori_loop(0, nvA, mk_wl(0, cntA), jnp.int32(0))
            wc = lax.fori_loop(0, nvB, mk_wl(b16, cntB), wc)
            wc = jnp.minimum(wc, WCAP)

            for g2 in range(16):
                i0 = g2 * 16
                dvv = wdest[pl.ds(i0, 16)]
                mval = (i0 + lane) < wc
                sidxm[g2, pl.ds(0, 16)] = jnp.where(
                    mval, dvv, TRASH + wid * 16 + lane)

            cbase = ct_lo * 128
            rowlo = slot * 32 + lane
            rowhi = rowlo + 16

            def ex_step(g, x):
                hv = whit[pl.ds(g * 16, 16)]
                colv = jnp.clip(hv - cbase, 0, CW * 128 - 1)
                for j in range(16):
                    cv = jnp.full((16,), colv[j], jnp.int32)
                    lo = plsc.load_gather(buf, [rowlo, cv])
                    hi = plsc.load_gather(buf, [rowhi, cv])
                    rows[g * 16 + j, pl.ds(0, 16)] = lo
                    rows[g * 16 + j, pl.ds(16, 16)] = hi
                return x

            lax.fori_loop(0, (wc + 15) >> 4, ex_step, 0)

            ng2 = (wc + 15) >> 4

            def sc_step(G, x):
                pltpu.async_copy(rows.at[pl.ds(G * 16, 16)],
                                 stg_h.at[sidxm.at[G]], sem_sc)
                return x

            lax.fori_loop(0, ng2, sc_step, 0)
            return ng2

        last_ng = lax.fori_loop(0, NCHK, chunk_body, jnp.int32(0))

        def drf(_, x):
            pltpu.make_async_copy(
                stg_h.at[pl.ds(0, 16)], rows.at[pl.ds(0, 16)],
                sem_sc).wait()
            return x
        lax.fori_loop(0, last_ng, drf, 0)

    with jax.named_scope("nodes_stream"):
        process_table(nodes_h, nh, nd, ncA, ncB, NHALF)
    with jax.named_scope("ctx_stream"):
        process_table(ctx_h, ch, cd, ccA, ccB, CHALF)


_kernel_a = functools.partial(
    pl.kernel,
    mesh=_MESH,
    out_type=jax.ShapeDtypeStruct((SROWS, 128), jnp.float32),
    scratch_types=[
        pltpu.VMEM((PIECE,), jnp.int32),
        pltpu.VMEM((PIECE,), jnp.int32),
        pltpu.VMEM((PIECE,), jnp.int32),
        pltpu.VMEM((PIECE,), jnp.int32),
        pltpu.VMEM((2 * NHALF + 32,), jnp.int32),
        pltpu.VMEM((2 * NHALF + 32,), jnp.int32),
        pltpu.VMEM((2 * CHALF + 32,), jnp.int32),
        pltpu.VMEM((2 * CHALF + 32,), jnp.int32),
        pltpu.VMEM((64, CW * 128), jnp.float32),
        pltpu.VMEM((WCAP + 16,), jnp.int32),
        pltpu.VMEM((WCAP + 16,), jnp.int32),
        pltpu.VMEM((16, 16), jnp.int32),
        pltpu.VMEM((WCAP, 128), jnp.float32),
        pltpu.SemaphoreType.DMA,
        pltpu.SemaphoreType.DMA,
        pltpu.SemaphoreType.DMA,
    ],
    compiler_params=_PARAMS,
)(_a_body)


def _b_body(stg_h, out_h, vib, vjb, ngb, dots, accv, sem):
    wid = lax.axis_index("s") * NC + lax.axis_index("c")
    lane = lax.iota(jnp.int32, 16)
    last = lane == 15
    SB = 128
    NSB = 512 // SB
    acc = jnp.zeros((16,), jnp.float32)

    def fire(s, d):
        base = wid * 512 + s * SB
        return [
            pltpu.async_copy(
                stg_h.at[pl.ds(OFF_VI + base, SB), pl.ds(0, 32)],
                vib.at[d], sem),
            pltpu.async_copy(
                stg_h.at[pl.ds(OFF_VJ + base, SB), pl.ds(0, 32)],
                vjb.at[d], sem),
            pltpu.async_copy(
                stg_h.at[pl.ds(OFF_NG + base * K, SB * K), pl.ds(0, 32)],
                ngb.at[d], sem),
        ]

    cps = fire(0, 0)
    for s in range(NSB):
        d = s % 2
        for c in cps:
            c.wait()
        if s + 1 < NSB:
            cps = fire(s + 1, 1 - d)
        vsel = vib.at[d]
        wsel = vjb.at[d]
        nsel = ngb.at[d]

        def bstep(b, ivec):
            vi0 = vsel[b, pl.ds(0, 16)]
            vi1 = vsel[b, pl.ds(16, 16)]
            vj0 = wsel[b, pl.ds(0, 16)]
            vj1 = wsel[b, pl.ds(16, 16)]
            cpos = plsc.cumsum(vi0 * vj0 + vi1 * vj1)
            plsc.store_scatter(dots, [ivec], cpos, mask=last)
            nvi0 = -vi0
            nvi1 = -vi1
            for k in range(K):
                n0 = nsel[b * K + k, pl.ds(0, 16)]
                n1 = nsel[b * K + k, pl.ds(16, 16)]
                cneg = plsc.cumsum(nvi0 * n0 + nvi1 * n1)
                plsc.store_scatter(dots, [ivec + (k + 1)], cneg, mask=last)
            return ivec + (K + 1)

        lax.fori_loop(0, SB, bstep, jnp.zeros((16,), jnp.int32))

        def sstep(j, a):
            dv = dots[pl.ds(j * 16, 16)]
            return a + 1.0 / (1.0 + jnp.exp(-dv))

        acc = lax.fori_loop(0, SB * (K + 1) // 16, sstep, acc)

    accv[pl.ds(0, 16)] = acc
    for j in range(1, 8):
        accv[pl.ds(j * 16, 16)] = jnp.zeros((16,), jnp.float32)
    pltpu.sync_copy(accv, out_h.at[wid])


_kernel_b = functools.partial(
    pl.kernel,
    mesh=_MESH,
    out_type=jax.ShapeDtypeStruct((NW, 128), jnp.float32),
    scratch_types=[
        pltpu.VMEM((2, 128, 32), jnp.float32),
        pltpu.VMEM((2, 128, 32), jnp.float32),
        pltpu.VMEM((2, 128 * K, 32), jnp.float32),
        pltpu.VMEM((128 * (K + 1),), jnp.float32),
        pltpu.VMEM((128,), jnp.float32),
        pltpu.SemaphoreType.DMA,
    ],
    compiler_params=pltpu.CompilerParams(
        needs_layout_passes=False, use_tc_tiling_on_sc=False),
)(_b_body)


def kernel(v_i, v_j, negsamples, nodes_embeddings, contextnodes_embeddings):
    nodes_t = nodes_embeddings.T
    ctx_t = contextnodes_embeddings.T
    vi = v_i.astype(jnp.int32)
    vj = v_j.astype(jnp.int32)
    ng = negsamples.astype(jnp.int32).reshape(-1)
    staging = _kernel_a(nodes_t, ctx_t, vi, vj, ng)
    partials = _kernel_b(staging)
    return -(jnp.sum(partials) / B)


# speedup vs baseline: 1.5298x; 1.0001x over previous
import functools

import jax
import jax.numpy as jnp
from jax import lax
from jax.experimental import pallas as pl
from jax.experimental.pallas import tpu as pltpu
from jax.experimental.pallas import tpu_sc as plsc

SIZE = 1000000
D = 32
B = 16384
K = 5
NC = 2
NS = 16
NW = NC * NS

TCOLS = (SIZE + 127) // 128
CW = 8
NCHK = 31
RANGE = NCHK * CW
PIECE = 2048

NHALF = 512
CHALF = 2560
WCAP = 256

SROWS = B * (K + 2) + NW * 16
TRASH = B * (K + 2)
OFF_VI = 0
OFF_VJ = B
OFF_NG = 2 * B

_MESH = plsc.VectorSubcoreMesh(core_axis_name="c", subcore_axis_name="s")
_PARAMS = pltpu.CompilerParams(needs_layout_passes=False)


def _a_body(nodes_h, ctx_h, vi_h, vj_h, ng_h, stg_h,
            ibuf0, ibuf1, ibuf2, ibuf3, nh, nd, ch, cd, buf, whit, wdest,
            sidxm, rows, sem_idx, sem_chunk, sem_sc):
    wid = lax.axis_index("s") * NC + lax.axis_index("c")
    lane = lax.iota(jnp.int32, 16)
    start = jnp.minimum(wid * TCOLS // NW, TCOLS - RANGE)

    def mk_scan2(bufA, bufB, hits_ref, dests_ref, half, boff):
        def step(k, carry):
            curA, curB, dvA, dvB = carry
            vA = bufA[pl.ds(k * 16, 16)]
            vB = bufB[pl.ds(k * 16, 16)]
            tA = vA >> 7
            tB = vB >> 7
            mA = (tA >= start) & (tA < start + RANGE)
            mB = (tB >= start) & (tB < start + RANGE)
            ccA = jnp.minimum(curA, half)
            ccB = boff + jnp.minimum(curB, half)
            plsc.store_compressed(hits_ref.at[pl.ds(ccA, 16)], vA, mask=mA)
            plsc.store_compressed(dests_ref.at[pl.ds(ccA, 16)], dvA, mask=mA)
            plsc.store_compressed(hits_ref.at[pl.ds(ccB, 16)], vB, mask=mB)
            plsc.store_compressed(dests_ref.at[pl.ds(ccB, 16)], dvB, mask=mB)
            nA = plsc.all_reduce_population_count(mA)
            nB = plsc.all_reduce_population_count(mB)
            return curA + nA[0], curB + nB[0], dvA + 16, dvB + 16
        return step

    def scan_input(src_h, n_words, dest_off, hits_ref, dests_ref, half,
                   curA, curB):
        npairs = n_words // (2 * PIECE)
        boff = half + 16
        cps = [pltpu.async_copy(src_h.at[pl.ds(0, PIECE)], ibuf0, sem_idx),
               pltpu.async_copy(src_h.at[pl.ds(PIECE, PIECE)], ibuf1, sem_idx)]
        for q in range(npairs):
            bA = ibuf0 if (q % 2 == 0) else ibuf2
            bB = ibuf1 if (q % 2 == 0) else ibuf3
            nA = ibuf2 if (q % 2 == 0) else ibuf0
            nB = ibuf3 if (q % 2 == 0) else ibuf1
            cps[0].wait()
            cps[1].wait()
            if q + 1 < npairs:
                w0 = (q + 1) * 2 * PIECE
                cps = [pltpu.async_copy(src_h.at[pl.ds(w0, PIECE)], nA,
                                        sem_idx),
                       pltpu.async_copy(src_h.at[pl.ds(w0 + PIECE, PIECE)],
                                        nB, sem_idx)]
            dvA = dest_off + q * 2 * PIECE + lane
            dvB = dvA + PIECE
            curA, curB, _, _ = lax.fori_loop(
                0, PIECE // 16,
                mk_scan2(bA, bB, hits_ref, dests_ref, half, boff),
                (curA, curB, dvA, dvB), unroll=2)
        return curA, curB

    with jax.named_scope("scan_phase"):
        ncA, ncB = scan_input(vi_h, B, OFF_VI, nh, nd, NHALF,
                              jnp.int32(0), jnp.int32(0))
        ccA, ccB = scan_input(vj_h, B, OFF_VJ, ch, cd, CHALF,
                              jnp.int32(0), jnp.int32(0))
        ccA, ccB = scan_input(ng_h, B * K, OFF_NG, ch, cd, CHALF, ccA, ccB)
        ncA = jnp.minimum(ncA, NHALF)
        ncB = jnp.minimum(ncB, NHALF)
        ccA = jnp.minimum(ccA, CHALF)
        ccB = jnp.minimum(ccB, CHALF)

    def fire_chunk(tbl_h, ci, slot):
        cb = (start + ci * CW) * 128
        pltpu.async_copy(
            tbl_h.at[pl.ds(0, 32), pl.ds(cb, CW * 128)],
            buf.at[pl.ds(slot * 32, 32)], sem_chunk)

    def process_table(tbl_h, hits_ref, dests_ref, cntA, cntB, half):
        nvA = (cntA + 15) >> 4
        nvB = (cntB + 15) >> 4
        b16 = (half + 16) // 16
        fire_chunk(tbl_h, 0, 0)

        def chunk_body(ci, prev_ng):
            slot = ci & 1
            @pl.when(ci + 1 < NCHK)
            def _():
                fire_chunk(tbl_h, ci + 1, 1 - slot)
            pltpu.make_async_copy(
                tbl_h.at[pl.ds(0, 32), pl.ds(0, CW * 128)],
                buf.at[pl.ds(slot * 32, 32)],
                sem_chunk).wait()
            def dr(_, x):
                pltpu.make_async_copy(
                    stg_h.at[pl.ds(0, 16)], rows.at[pl.ds(0, 16)],
                    sem_sc).wait()
                return x
            lax.fori_loop(0, prev_ng, dr, 0)

            ct_lo = start + ci * CW

            def mk_wl(base16, cnt):
                def wl_step(k, wc):
                    v = hits_ref[pl.ds((base16 + k) * 16, 16)]
                    dvv = dests_ref[pl.ds((base16 + k) * 16, 16)]
                    t = v >> 7
                    m = (((k * 16 + lane) < cnt)
                         & (t >= ct_lo) & (t < ct_lo + CW))
                    wcc = jnp.minimum(wc, WCAP)
                    plsc.store_compressed(whit.at[pl.ds(wcc, 16)], v, mask=m)
                    plsc.store_compressed(wdest.at[pl.ds(wcc, 16)], dvv,
                                          mask=m)
                    return wc + plsc.all_reduce_population_count(m)[0]
                return wl_step

            wc = lax.fori_loop(0, nvA, mk_wl(0, cntA), jnp.int32(0))
            wc = lax.fori_loop(0, nvB, mk_wl(b16, cntB), wc)
            wc = jnp.minimum(wc, WCAP)

            for g2 in range(16):
                i0 = g2 * 16
                dvv = wdest[pl.ds(i0, 16)]
                mval = (i0 + lane) < wc
                sidxm[g2, pl.ds(0, 16)] = jnp.where(
                    mval, dvv, TRASH + wid * 16 + lane)

            cbase = ct_lo * 128
            rowlo = slot * 32 + lane
            rowhi = rowlo + 16

            def ex_step(g, x):
                hv = whit[pl.ds(g * 16, 16)]
                colv = jnp.clip(hv - cbase, 0, CW * 128 - 1)
                for j in range(16):
                    cv = jnp.full((16,), colv[j], jnp.int32)
                    lo = plsc.load_gather(buf, [rowlo, cv])
                    hi = plsc.load_gather(buf, [rowhi, cv])
                    rows[g * 16 + j, pl.ds(0, 16)] = lo
                    rows[g * 16 + j, pl.ds(16, 16)] = hi
                return x

            lax.fori_loop(0, (wc + 15) >> 4, ex_step, 0)

            ng2 = (wc + 15) >> 4

            def sc_step(G, x):
                pltpu.async_copy(rows.at[pl.ds(G * 16, 16)],
                                 stg_h.at[sidxm.at[G]], sem_sc)
                return x

            lax.fori_loop(0, ng2, sc_step, 0)
            return ng2

        last_ng = lax.fori_loop(0, NCHK, chunk_body, jnp.int32(0))

        def drf(_, x):
            pltpu.make_async_copy(
                stg_h.at[pl.ds(0, 16)], rows.at[pl.ds(0, 16)],
                sem_sc).wait()
            return x
        lax.fori_loop(0, last_ng, drf, 0)

    with jax.named_scope("nodes_stream"):
        process_table(nodes_h, nh, nd, ncA, ncB, NHALF)
    with jax.named_scope("ctx_stream"):
        process_table(ctx_h, ch, cd, ccA, ccB, CHALF)


_kernel_a = functools.partial(
    pl.kernel,
    mesh=_MESH,
    out_type=jax.ShapeDtypeStruct((SROWS, 128), jnp.float32),
    scratch_types=[
        pltpu.VMEM((PIECE,), jnp.int32),
        pltpu.VMEM((PIECE,), jnp.int32),
        pltpu.VMEM((PIECE,), jnp.int32),
        pltpu.VMEM((PIECE,), jnp.int32),
        pltpu.VMEM((2 * NHALF + 32,), jnp.int32),
        pltpu.VMEM((2 * NHALF + 32,), jnp.int32),
        pltpu.VMEM((2 * CHALF + 32,), jnp.int32),
        pltpu.VMEM((2 * CHALF + 32,), jnp.int32),
        pltpu.VMEM((64, CW * 128), jnp.float32),
        pltpu.VMEM((WCAP + 16,), jnp.int32),
        pltpu.VMEM((WCAP + 16,), jnp.int32),
        pltpu.VMEM((16, 16), jnp.int32),
        pltpu.VMEM((WCAP, 128), jnp.float32),
        pltpu.SemaphoreType.DMA,
        pltpu.SemaphoreType.DMA,
        pltpu.SemaphoreType.DMA,
    ],
    compiler_params=_PARAMS,
)(_a_body)


def _b_body(stg_h, out_h, vib, vjb, ngb, dots, accv, sem):
    wid = lax.axis_index("s") * NC + lax.axis_index("c")
    lane = lax.iota(jnp.int32, 16)
    last = lane == 15
    SB = 128
    NSB = 512 // SB
    acc = jnp.zeros((16,), jnp.float32)

    def fire(s, d):
        base = wid * 512 + s * SB
        return [
            pltpu.async_copy(
                stg_h.at[pl.ds(OFF_VI + base, SB), pl.ds(0, 32)],
                vib.at[d], sem),
            pltpu.async_copy(
                stg_h.at[pl.ds(OFF_VJ + base, SB), pl.ds(0, 32)],
                vjb.at[d], sem),
            pltpu.async_copy(
                stg_h.at[pl.ds(OFF_NG + base * K, SB * K), pl.ds(0, 32)],
                ngb.at[d], sem),
        ]

    cps = fire(0, 0)
    for s in range(NSB):
        d = s % 2
        for c in cps:
            c.wait()
        if s + 1 < NSB:
            cps = fire(s + 1, 1 - d)
        vsel = vib.at[d]
        wsel = vjb.at[d]
        nsel = ngb.at[d]

        def bstep(b, ivec):
            vi0 = vsel[b, pl.ds(0, 16)]
            vi1 = vsel[b, pl.ds(16, 16)]
            vj0 = wsel[b, pl.ds(0, 16)]
            vj1 = wsel[b, pl.ds(16, 16)]
            cpos = plsc.cumsum(vi0 * vj0 + vi1 * vj1)
            plsc.store_scatter(dots, [ivec], cpos, mask=last)
            nvi0 = -vi0
            nvi1 = -vi1
            for k in range(K):
                n0 = nsel[b * K + k, pl.ds(0, 16)]
                n1 = nsel[b * K + k, pl.ds(16, 16)]
                cneg = plsc.cumsum(nvi0 * n0 + nvi1 * n1)
                plsc.store_scatter(dots, [ivec + (k + 1)], cneg, mask=last)
            return ivec + (K + 1)

        lax.fori_loop(0, SB, bstep, jnp.zeros((16,), jnp.int32))

        def sstep(j, a):
            dv = dots[pl.ds(j * 16, 16)]
            return a + 1.0 / (1.0 + jnp.exp(-dv))

        acc = lax.fori_loop(0, SB * (K + 1) // 16, sstep, acc)

    accv[pl.ds(0, 16)] = acc
    for j in range(1, 8):
        accv[pl.ds(j * 16, 16)] = jnp.zeros((16,), jnp.float32)
    pltpu.sync_copy(accv, out_h.at[wid])


_kernel_b = functools.partial(
    pl.kernel,
    mesh=_MESH,
    out_type=jax.ShapeDtypeStruct((NW, 128), jnp.float32),
    scratch_types=[
        pltpu.VMEM((2, 128, 32), jnp.float32),
        pltpu.VMEM((2, 128, 32), jnp.float32),
        pltpu.VMEM((2, 128 * K, 32), jnp.float32),
        pltpu.VMEM((128 * (K + 1),), jnp.float32),
        pltpu.VMEM((128,), jnp.float32),
        pltpu.SemaphoreType.DMA,
    ],
    compiler_params=pltpu.CompilerParams(
        needs_layout_passes=False, use_tc_tiling_on_sc=False),
)(_b_body)


def kernel(v_i, v_j, negsamples, nodes_embeddings, contextnodes_embeddings):
    nodes_t = nodes_embeddings.T
    ctx_t = contextnodes_embeddings.T
    vi = v_i.astype(jnp.int32)
    vj = v_j.astype(jnp.int32)
    ng = negsamples.astype(jnp.int32).reshape(-1)
    staging = _kernel_a(nodes_t, ctx_t, vi, vj, ng)
    partials = _kernel_b(staging)
    return -(jnp.sum(partials) / B)
